# Initial kernel scaffold; baseline (speedup 1.0000x reference)
import jax
import jax.numpy as jnp
from jax.experimental import pallas as pl


def kernel(x, tok_table, pos_table):
    raise NotImplementedError("write your pallas kernel here")


# speedup vs baseline: 1.0044x; 1.0044x over previous
import functools

import jax
import jax.numpy as jnp
from jax import lax
from jax.experimental import pallas as pl
from jax.experimental.pallas import tpu as pltpu
from jax.experimental.pallas import tpu_sc as plsc

VOCAB = 100000
EMBED = 128
NPOS = 2048
B = 4
S = 2048

NC = 2
NS = 16
NW = NC * NS
NROWS = B * S
ROWS_PER_W = NROWS // NW
IDX_CHUNKS = ROWS_PER_W // 128
LANES = 16

_mesh = plsc.VectorSubcoreMesh(
    core_axis_name="c", subcore_axis_name="s", num_cores=NC, num_subcores=NS
)


@functools.partial(
    pl.kernel,
    out_type=jax.ShapeDtypeStruct((NROWS, EMBED), jnp.float32),
    mesh=_mesh,
    scratch_types=[
        pltpu.VMEM((IDX_CHUNKS, 128), jnp.int32),
        pltpu.VMEM((ROWS_PER_W, EMBED), jnp.float32),
        pltpu.VMEM((ROWS_PER_W, EMBED), jnp.float32),
        pltpu.SemaphoreType.DMA,
    ],
)
def _embed_kernel(x_hbm, tok_hbm, pos_hbm, out_hbm, idx_v, tok_v, pos_v, sem):
    wid = lax.axis_index("s") * NC + lax.axis_index("c")
    base = wid * ROWS_PER_W
    pos_base = lax.rem(base, S)

    pltpu.sync_copy(x_hbm.at[wid], idx_v)
    copies = []
    for j in range(IDX_CHUNKS):
        copies.append(
            pltpu.async_copy(
                tok_hbm.at[idx_v.at[j]], tok_v.at[pl.ds(j * 128, 128)], sem
            )
        )
    copies.append(
        pltpu.async_copy(pos_hbm.at[pl.ds(pos_base, ROWS_PER_W)], pos_v, sem)
    )
    for cp in copies:
        cp.wait()

    def add_row(r, carry):
        for c in range(EMBED // LANES):
            sl = pl.ds(c * LANES, LANES)
            tok_v[r, sl] = tok_v[r, sl] + pos_v[r, sl]
        return carry

    lax.fori_loop(0, ROWS_PER_W, add_row, 0, unroll=2)

    pltpu.sync_copy(tok_v, out_hbm.at[pl.ds(base, ROWS_PER_W)])


def kernel(x, tok_table, pos_table):
    x2 = x.reshape(NW, IDX_CHUNKS, 128)
    out = _embed_kernel(x2, tok_table, pos_table)
    return out.reshape(B, S, EMBED)


# speedup vs baseline: 1.1392x; 1.1343x over previous
import functools

import jax
import jax.numpy as jnp
from jax import lax
from jax.experimental import pallas as pl
from jax.experimental.pallas import tpu as pltpu
from jax.experimental.pallas import tpu_sc as plsc

VOCAB = 100000
EMBED = 128
NPOS = 2048
B = 4
S = 2048

NC = 2
NS = 16
NW = NC * NS
NROWS = B * S
ROWS_PER_W = NROWS // NW
WPB = S // ROWS_PER_W
NCHUNK = 4
CH = ROWS_PER_W // NCHUNK
LANES = 16

_mesh = plsc.VectorSubcoreMesh(
    core_axis_name="c", subcore_axis_name="s", num_cores=NC, num_subcores=NS
)


@functools.partial(
    pl.kernel,
    out_type=jax.ShapeDtypeStruct((B, S, EMBED), jnp.float32),
    mesh=_mesh,
    scratch_types=[
        pltpu.VMEM((NCHUNK, CH), jnp.int32),
        pltpu.VMEM((ROWS_PER_W, EMBED), jnp.float32),
        pltpu.VMEM((ROWS_PER_W, EMBED), jnp.float32),
        pltpu.SemaphoreType.DMA,
        pltpu.SemaphoreType.DMA,
        pltpu.SemaphoreType.DMA,
        pltpu.SemaphoreType.DMA,
        pltpu.SemaphoreType.DMA,
        pltpu.SemaphoreType.DMA,
    ],
)
def _embed_kernel(x_hbm, tok_hbm, pos_hbm, out_hbm, idx_v, tok_v, pos_v,
                  sem_in, sem0, sem1, sem2, sem3, sem_out):
    sems = [sem0, sem1, sem2, sem3]
    wid = lax.axis_index("s") * NC + lax.axis_index("c")
    b = wid // WPB
    s0 = lax.rem(wid, WPB) * ROWS_PER_W

    idx_cps = [
        pltpu.async_copy(x_hbm.at[b, pl.ds(s0 + c * CH, CH)], idx_v.at[c], sem_in)
        for c in range(NCHUNK)
    ]
    pos_cps = [
        pltpu.async_copy(
            pos_hbm.at[pl.ds(s0 + c * CH, CH)], pos_v.at[pl.ds(c * CH, CH)], sems[c]
        )
        for c in range(NCHUNK)
    ]
    for cp in idx_cps:
        cp.wait()
    g_cps = [
        pltpu.async_copy(
            tok_hbm.at[idx_v.at[c]], tok_v.at[pl.ds(c * CH, CH)], sems[c]
        )
        for c in range(NCHUNK)
    ]

    out_cps = []
    for c in range(NCHUNK):
        pos_cps[c].wait()
        g_cps[c].wait()

        def add_row(r, carry):
            for k in range(EMBED // LANES):
                sl = pl.ds(k * LANES, LANES)
                tok_v[r, sl] = tok_v[r, sl] + pos_v[r, sl]
            return carry

        lax.fori_loop(c * CH, (c + 1) * CH, add_row, 0, unroll=2)
        out_cps.append(
            pltpu.async_copy(
                tok_v.at[pl.ds(c * CH, CH)],
                out_hbm.at[b, pl.ds(s0 + c * CH, CH)],
                sem_out,
            )
        )
    for cp in out_cps:
        cp.wait()


def kernel(x, tok_table, pos_table):
    return _embed_kernel(x, tok_table, pos_table)


# speedup vs baseline: 1.3201x; 1.1588x over previous
import functools

import jax
import jax.numpy as jnp
from jax import lax
from jax.experimental import pallas as pl
from jax.experimental.pallas import tpu as pltpu
from jax.experimental.pallas import tpu_sc as plsc

VOCAB = 100000
EMBED = 128
NPOS = 2048
B = 4
S = 2048

NC = 2
NS = 16
NW = NC * NS
NROWS = B * S
ROWS_PER_W = NROWS // NW
WPB = S // ROWS_PER_W
NCHUNK = 4
CH = ROWS_PER_W // NCHUNK
LANES = 16

_mesh = plsc.VectorSubcoreMesh(
    core_axis_name="c", subcore_axis_name="s", num_cores=NC, num_subcores=NS
)


@functools.partial(
    pl.kernel,
    out_type=jax.ShapeDtypeStruct((B, S, EMBED), jnp.float32),
    mesh=_mesh,
    scratch_types=[
        pltpu.VMEM((NCHUNK, CH), jnp.int32),
        pltpu.VMEM((NCHUNK, CH), jnp.int32),
        pltpu.VMEM((ROWS_PER_W, EMBED), jnp.float32),
        pltpu.VMEM_SHARED((NS * ROWS_PER_W, EMBED), jnp.float32),
        pltpu.SemaphoreType.DMA,
        pltpu.SemaphoreType.DMA,
        pltpu.SemaphoreType.DMA,
        pltpu.SemaphoreType.DMA,
        pltpu.SemaphoreType.DMA,
        pltpu.SemaphoreType.DMA,
    ],
)
def _embed_kernel(x_hbm, tok_hbm, pos_hbm, out_hbm, idx_v, ids_v, tok_v,
                  acc_sh, sem_in, sem0, sem1, sem2, sem3, sem_out):
    sems = [sem0, sem1, sem2, sem3]
    cid = lax.axis_index("c")
    sid = lax.axis_index("s")
    wid = sid * NC + cid
    b = wid // WPB
    s0 = lax.rem(wid, WPB) * ROWS_PER_W
    spbase = sid * ROWS_PER_W

    idx_cps = [
        pltpu.async_copy(x_hbm.at[b, pl.ds(s0 + c * CH, CH)], idx_v.at[c], sem_in)
        for c in range(NCHUNK)
    ]
    pos_cps = [
        pltpu.async_copy(
            pos_hbm.at[pl.ds(s0 + c * CH, CH)],
            acc_sh.at[pl.ds(spbase + c * CH, CH)],
            sems[c],
        )
        for c in range(NCHUNK)
    ]

    lane = lax.iota(jnp.int32, 16)
    for j in range(NCHUNK):
        for k in range(CH // LANES):
            ids_v[j, pl.ds(k * LANES, LANES)] = lane + (
                spbase + j * CH + k * LANES
            )

    for cp in idx_cps:
        cp.wait()
    g_cps = [
        pltpu.async_copy(
            tok_hbm.at[idx_v.at[c]], tok_v.at[pl.ds(c * CH, CH)], sems[c]
        )
        for c in range(NCHUNK)
    ]

    sa_cps = []
    for c in range(NCHUNK):
        pos_cps[c].wait()
        g_cps[c].wait()
        sa_cps.append(
            pltpu.async_copy(
                tok_v.at[pl.ds(c * CH, CH)],
                acc_sh.at[ids_v.at[c]],
                sems[c],
                add=True,
            )
        )
    out_cps = []
    for c in range(NCHUNK):
        sa_cps[c].wait()
        out_cps.append(
            pltpu.async_copy(
                acc_sh.at[pl.ds(spbase + c * CH, CH)],
                out_hbm.at[b, pl.ds(s0 + c * CH, CH)],
                sem_out,
            )
        )
    for cp in out_cps:
        cp.wait()


def kernel(x, tok_table, pos_table):
    return _embed_kernel(x, tok_table, pos_table)


# speedup vs baseline: 1.3337x; 1.0103x over previous
import functools

import jax
import jax.numpy as jnp
from jax import lax
from jax.experimental import pallas as pl
from jax.experimental.pallas import tpu as pltpu
from jax.experimental.pallas import tpu_sc as plsc

VOCAB = 100000
EMBED = 128
NPOS = 2048
B = 4
S = 2048

NC = 2
NS = 16
NW = NC * NS
NROWS = B * S
ROWS_PER_W = NROWS // NW
WPB = S // ROWS_PER_W
NCHUNK = 8
CH = ROWS_PER_W // NCHUNK
LANES = 16

_mesh = plsc.VectorSubcoreMesh(
    core_axis_name="c", subcore_axis_name="s", num_cores=NC, num_subcores=NS
)


@functools.partial(
    pl.kernel,
    out_type=jax.ShapeDtypeStruct((B, S, EMBED), jnp.float32),
    mesh=_mesh,
    scratch_types=[
        pltpu.VMEM((ROWS_PER_W,), jnp.int32),
        pltpu.VMEM((NCHUNK, CH), jnp.int32),
        pltpu.VMEM((ROWS_PER_W, EMBED), jnp.float32),
        pltpu.VMEM_SHARED((NS * ROWS_PER_W, EMBED), jnp.float32),
        pltpu.SemaphoreType.DMA,
        pltpu.SemaphoreType.DMA,
        pltpu.SemaphoreType.DMA,
        pltpu.SemaphoreType.DMA,
        pltpu.SemaphoreType.DMA,
        pltpu.SemaphoreType.DMA,
        pltpu.SemaphoreType.DMA,
        pltpu.SemaphoreType.DMA,
        pltpu.SemaphoreType.DMA,
        pltpu.SemaphoreType.DMA,
    ],
)
def _embed_kernel(x_hbm, tok_hbm, pos_hbm, out_hbm, idx_v, ids_v, tok_v,
                  acc_sh, sem_in, sem0, sem1, sem2, sem3, sem4, sem5, sem6,
                  sem7, sem_out):
    sems = [sem0, sem1, sem2, sem3, sem4, sem5, sem6, sem7]
    cid = lax.axis_index("c")
    sid = lax.axis_index("s")
    wid = sid * NC + cid
    b = wid // WPB
    s0 = lax.rem(wid, WPB) * ROWS_PER_W
    spbase = sid * ROWS_PER_W

    idx_cp = pltpu.async_copy(x_hbm.at[b, pl.ds(s0, ROWS_PER_W)], idx_v, sem_in)
    pos_cps = [
        pltpu.async_copy(
            pos_hbm.at[pl.ds(s0 + c * CH, CH)],
            acc_sh.at[pl.ds(spbase + c * CH, CH)],
            sems[c],
        )
        for c in range(NCHUNK)
    ]

    lane = lax.iota(jnp.int32, 16)
    for j in range(NCHUNK):
        for k in range(CH // LANES):
            ids_v[j, pl.ds(k * LANES, LANES)] = lane + (
                spbase + j * CH + k * LANES
            )

    idx_cp.wait()
    g_cps = [
        pltpu.async_copy(
            tok_hbm.at[idx_v.at[pl.ds(c * CH, CH)]],
            tok_v.at[pl.ds(c * CH, CH)],
            sems[c],
        )
        for c in range(NCHUNK)
    ]

    sa_cps = []
    for c in range(NCHUNK):
        pos_cps[c].wait()
        g_cps[c].wait()
        sa_cps.append(
            pltpu.async_copy(
                tok_v.at[pl.ds(c * CH, CH)],
                acc_sh.at[ids_v.at[c]],
                sems[c],
                add=True,
            )
        )
        if c >= 1:
            sa_cps[c - 1].wait()
            pltpu.async_copy(
                acc_sh.at[pl.ds(spbase + (c - 1) * CH, CH)],
                out_hbm.at[b, pl.ds(s0 + (c - 1) * CH, CH)],
                sem_out,
            )
    sa_cps[NCHUNK - 1].wait()
    last_out = pltpu.async_copy(
        acc_sh.at[pl.ds(spbase + (NCHUNK - 1) * CH, CH)],
        out_hbm.at[b, pl.ds(s0 + (NCHUNK - 1) * CH, CH)],
        sem_out,
    )
    for _ in range(NCHUNK):
        last_out.wait()


def kernel(x, tok_table, pos_table):
    return _embed_kernel(x, tok_table, pos_table)
